# Initial kernel scaffold; baseline (speedup 1.0000x reference)
import jax
import jax.numpy as jnp
from jax.experimental import pallas as pl


def kernel(entities, relations, edge_index):
    raise NotImplementedError("write your pallas kernel here")


# speedup vs baseline: 3.7786x; 3.7786x over previous
import functools

import jax
import jax.numpy as jnp
from jax import lax
from jax.experimental import pallas as pl
from jax.experimental.pallas import tpu as pltpu
from jax.experimental.pallas import tpu_sc as plsc

N = 10000
E = 320000
D = 128
DH = 64
NC = 2
NS = 16
L = 16
CW = 16
B = 80
ET = E // NS
NCHUNK = ET // B
NPT = N // NS


def _body(eL, eR, src, dst, out, sums_sp, cnts_sp, big_v, cnt_v, gbuf, sidx,
          didx, ones_v, sem):
    cid = lax.axis_index("c")
    sid = lax.axis_index("s")
    nb = sid * NPT

    zero16 = jnp.zeros((L,), jnp.float32)
    one16 = jnp.ones((L,), jnp.float32)

    def zrow(i, c):
        for j in range(DH // L):
            big_v[i, pl.ds(j * L, L)] = zero16
        cnt_v[i, :] = zero16
        return c

    lax.fori_loop(0, NPT, zrow, 0)

    def orow(i, c):
        ones_v[i, :] = one16
        return c

    lax.fori_loop(0, B, orow, 0)

    pltpu.sync_copy(big_v, sums_sp.at[pl.ds(nb, NPT)])
    pltpu.sync_copy(cnt_v, cnts_sp.at[pl.ds(nb, NPT)])
    plsc.subcore_barrier()

    def phase1(ent_hbm):
        ebase = sid * ET

        def chunk(k, c):
            eb = ebase + k * B
            pltpu.sync_copy(src.at[pl.ds(eb, B)], sidx)
            pltpu.sync_copy(dst.at[pl.ds(eb, B)], didx)
            pltpu.async_copy(ent_hbm.at[sidx], gbuf, sem).wait()
            pltpu.sync_copy(gbuf, sums_sp.at[didx], add=True)
            pltpu.sync_copy(ones_v, cnts_sp.at[didx], add=True)
            return c

        lax.fori_loop(0, NCHUNK, chunk, 0)

    @pl.when(cid == 0)
    def _():
        phase1(eL)

    @pl.when(cid != 0)
    def _():
        phase1(eR)

    plsc.subcore_barrier()

    pltpu.sync_copy(sums_sp.at[pl.ds(nb, NPT)], big_v)
    pltpu.sync_copy(cnts_sp.at[pl.ds(nb, NPT)], cnt_v)

    def norm(i, c):
        cnt = cnt_v[i, :]
        inv = 1.0 / jnp.maximum(cnt, 1.0)
        for j in range(DH // L):
            sl = pl.ds(j * L, L)
            big_v[i, sl] = big_v[i, sl] * inv
        return c

    lax.fori_loop(0, NPT, norm, 0)
    pltpu.sync_copy(big_v, out.at[cid, sid])


_sc_call = pl.kernel(
    _body,
    out_type=jax.ShapeDtypeStruct((NC, NS, NPT, DH), jnp.float32),
    mesh=plsc.VectorSubcoreMesh(core_axis_name="c", subcore_axis_name="s"),
    compiler_params=pltpu.CompilerParams(use_tc_tiling_on_sc=False),
    scratch_types=[
        pltpu.VMEM_SHARED((N, DH), jnp.float32),
        pltpu.VMEM_SHARED((N, CW), jnp.float32),
        pltpu.VMEM((NPT, DH), jnp.float32),
        pltpu.VMEM((NPT, CW), jnp.float32),
        pltpu.VMEM((B, DH), jnp.float32),
        pltpu.VMEM((B,), jnp.int32),
        pltpu.VMEM((B,), jnp.int32),
        pltpu.VMEM((B, CW), jnp.float32),
        pltpu.SemaphoreType.DMA,
    ],
)


@jax.jit
def kernel(entities, relations, edge_index):
    del relations
    halves = _sc_call(entities[:, :DH], entities[:, DH:], edge_index[0],
                      edge_index[2]).reshape(NC, N, DH)
    return jnp.concatenate([halves[0], halves[1]], axis=1)
